# baseline (speedup 1.0000x reference)
import functools

import jax
import jax.numpy as jnp
from jax.experimental import pallas as pl
from jax.experimental.pallas import tpu as pltpu
from jax.experimental.pallas import tpu_sc as plsc

_D = 384
_COSTS = (0.05, 0.25, 0.6)
_NC, _NS = 2, 16
_NW = _NC * _NS


def _vq_body(x_ref, cb_ref, idx_ref, loss_ref, *, n_codes):
    x = x_ref[...]
    cb = cb_ref[...]
    x2 = jnp.sum(x * x, axis=1, keepdims=True)
    cb2 = jnp.sum(cb * cb, axis=1)[None, :]
    xc = jax.lax.dot_general(x, cb, (((1,), (1,)), ((), ())),
                             preferred_element_type=jnp.float32)
    d2 = x2 - 2.0 * xc + cb2
    m = jnp.min(d2, axis=1, keepdims=True)
    iota = jax.lax.broadcasted_iota(jnp.int32, d2.shape, 1)
    idx = jnp.min(jnp.where(d2 == m, iota, n_codes), axis=1)
    idx_ref[...] = idx
    s = jnp.sum(m)

    @pl.when(pl.program_id(0) == 0)
    def _init():
        loss_ref[0, 0] = 0.0

    loss_ref[0, 0] += s


def _vq_level(x_flat, cb, block_rows):
    n, d = x_flat.shape
    k = cb.shape[0]
    grid = n // block_rows
    body = functools.partial(_vq_body, n_codes=k)
    idx, loss_sum = pl.pallas_call(
        body,
        grid=(grid,),
        in_specs=[
            pl.BlockSpec((block_rows, d), lambda i: (i, 0)),
            pl.BlockSpec((k, d), lambda i: (0, 0)),
        ],
        out_specs=[
            pl.BlockSpec((block_rows,), lambda i: (i,)),
            pl.BlockSpec((1, 1), lambda i: (0, 0), memory_space=pltpu.SMEM),
        ],
        out_shape=[
            jax.ShapeDtypeStruct((n,), jnp.int32),
            jax.ShapeDtypeStruct((1, 1), jnp.float32),
        ],
    )(x_flat, cb)
    return idx, loss_sum[0, 0]


def _sc_gather(cb, idx, n_rows, chunk):
    rpw = n_rows // _NW
    nchunks = rpw // chunk
    mesh = plsc.VectorSubcoreMesh(
        core_axis_name="c", subcore_axis_name="s",
        num_cores=_NC, num_subcores=_NS)

    @functools.partial(
        pl.kernel,
        out_type=jax.ShapeDtypeStruct((n_rows, _D), jnp.float32),
        mesh=mesh,
        scratch_types=[
            pltpu.VMEM((rpw,), jnp.int32),
            pltpu.VMEM((chunk, _D), jnp.float32),
            pltpu.VMEM((chunk, _D), jnp.float32),
            pltpu.SemaphoreType.DMA,
            pltpu.SemaphoreType.DMA,
        ],
    )
    def gather_kernel(cb_hbm, idx_hbm, out_hbm, idx_v, rows_a, rows_b, gsem, osem):
        wid = jax.lax.axis_index("s") * _NC + jax.lax.axis_index("c")
        base = wid * rpw
        pltpu.sync_copy(idx_hbm.at[pl.ds(base, rpw)], idx_v)
        bufs = (rows_a, rows_b)
        g_descs = [None] * nchunks
        o_descs = [None] * nchunks
        g_descs[0] = pltpu.async_copy(
            cb_hbm.at[idx_v.at[pl.ds(0, chunk)]], bufs[0], gsem)
        for c in range(nchunks):
            if c + 1 < nchunks:
                if c >= 1:
                    o_descs[c - 1].wait()
                g_descs[c + 1] = pltpu.async_copy(
                    cb_hbm.at[idx_v.at[pl.ds((c + 1) * chunk, chunk)]],
                    bufs[(c + 1) % 2], gsem)
            g_descs[c].wait()
            o_descs[c] = pltpu.async_copy(
                bufs[c % 2], out_hbm.at[pl.ds(base + c * chunk, chunk)], osem)
        if nchunks >= 2:
            o_descs[nchunks - 2].wait()
        o_descs[nchunks - 1].wait()

    return gather_kernel(cb, idx)


def kernel(l0, l1, l2, cb0, cb1, cb2):
    levels = ((l0, cb0, 1024, 64), (l1, cb1, 1024, 128), (l2, cb2, 1024, 128))
    idxs, qs, sums = [], [], []
    for x, cb, br, chunk in levels:
        xf = x.reshape(-1, _D)
        idx, s = _vq_level(xf, cb, br)
        q = _sc_gather(cb, idx, xf.shape[0], chunk)
        idxs.append(idx.reshape(x.shape[:-1]))
        qs.append(q.reshape(x.shape))
        sums.append(s)
    total = (
        (1.0 + _COSTS[0]) * sums[0] / l0.size
        + (1.0 + _COSTS[1]) * sums[1] / l1.size
        + (1.0 + _COSTS[2]) * sums[2] / l2.size
    )
    return (idxs[0], idxs[1], idxs[2], total, qs[0], qs[1], qs[2])


# speedup vs baseline: 1.5199x; 1.5199x over previous
import functools

import jax
import jax.numpy as jnp
from jax.experimental import pallas as pl
from jax.experimental.pallas import tpu as pltpu
from jax.experimental.pallas import tpu_sc as plsc

_D = 384
_COSTS = (0.05, 0.25, 0.6)
_NC, _NS = 2, 16
_NW = _NC * _NS


def _vq_body(x_ref, cb_ref, idx_ref, loss_ref, *, n_codes):
    x = x_ref[...]
    cb = cb_ref[...]
    x2 = jnp.sum(x * x, axis=1, keepdims=True)
    cb2 = jnp.sum(cb * cb, axis=1)[None, :]
    xc = jax.lax.dot_general(x, cb, (((1,), (1,)), ((), ())),
                             preferred_element_type=jnp.float32)
    d2 = x2 - 2.0 * xc + cb2
    m = jnp.min(d2, axis=1, keepdims=True)
    iota = jax.lax.broadcasted_iota(jnp.int32, d2.shape, 1)
    idx = jnp.min(jnp.where(d2 == m, iota, n_codes), axis=1)
    idx_ref[...] = idx
    s = jnp.sum(m)

    @pl.when(pl.program_id(0) == 0)
    def _init():
        loss_ref[0, 0] = 0.0

    loss_ref[0, 0] += s


def _vq_body_q(x_ref, cb_ref, idx_ref, loss_ref, q_ref, *, n_codes):
    x = x_ref[...]
    cb = cb_ref[...]
    x2 = jnp.sum(x * x, axis=1, keepdims=True)
    cb2 = jnp.sum(cb * cb, axis=1)[None, :]
    xc = jax.lax.dot_general(x, cb, (((1,), (1,)), ((), ())),
                             preferred_element_type=jnp.float32)
    d2 = x2 - 2.0 * xc + cb2
    m = jnp.min(d2, axis=1, keepdims=True)
    iota = jax.lax.broadcasted_iota(jnp.int32, d2.shape, 1)
    idx = jnp.min(jnp.where(d2 == m, iota, n_codes), axis=1)
    idx_ref[...] = idx
    onehot = (iota == idx[:, None]).astype(jnp.float32)
    q_ref[...] = jax.lax.dot_general(
        onehot, cb, (((1,), (0,)), ((), ())),
        preferred_element_type=jnp.float32,
        precision=jax.lax.Precision.HIGHEST)
    s = jnp.sum(m)

    @pl.when(pl.program_id(0) == 0)
    def _init():
        loss_ref[0, 0] = 0.0

    loss_ref[0, 0] += s


def _vq_level(x_flat, cb, block_rows, with_q):
    n, d = x_flat.shape
    k = cb.shape[0]
    grid = n // block_rows
    out_specs = [
        pl.BlockSpec((block_rows,), lambda i: (i,)),
        pl.BlockSpec((1, 1), lambda i: (0, 0), memory_space=pltpu.SMEM),
    ]
    out_shape = [
        jax.ShapeDtypeStruct((n,), jnp.int32),
        jax.ShapeDtypeStruct((1, 1), jnp.float32),
    ]
    if with_q:
        body = functools.partial(_vq_body_q, n_codes=k)
        out_specs.append(pl.BlockSpec((block_rows, d), lambda i: (i, 0)))
        out_shape.append(jax.ShapeDtypeStruct((n, d), jnp.float32))
    else:
        body = functools.partial(_vq_body, n_codes=k)
    outs = pl.pallas_call(
        body,
        grid=(grid,),
        in_specs=[
            pl.BlockSpec((block_rows, d), lambda i: (i, 0)),
            pl.BlockSpec((k, d), lambda i: (0, 0)),
        ],
        out_specs=out_specs,
        out_shape=out_shape,
    )(x_flat, cb)
    if with_q:
        idx, loss_sum, q = outs
        return idx, loss_sum[0, 0], q
    idx, loss_sum = outs
    return idx, loss_sum[0, 0], None


def _sc_gather(cb, idx, n_rows, chunk):
    rpw = n_rows // _NW
    nchunks = rpw // chunk
    mesh = plsc.VectorSubcoreMesh(
        core_axis_name="c", subcore_axis_name="s",
        num_cores=_NC, num_subcores=_NS)

    @functools.partial(
        pl.kernel,
        out_type=jax.ShapeDtypeStruct((n_rows, _D), jnp.float32),
        mesh=mesh,
        scratch_types=[
            pltpu.VMEM((rpw,), jnp.int32),
            pltpu.VMEM((chunk, _D), jnp.float32),
            pltpu.VMEM((chunk, _D), jnp.float32),
            pltpu.SemaphoreType.DMA,
            pltpu.SemaphoreType.DMA,
        ],
    )
    def gather_kernel(cb_hbm, idx_hbm, out_hbm, idx_v, rows_a, rows_b, gsem, osem):
        wid = jax.lax.axis_index("s") * _NC + jax.lax.axis_index("c")
        base = wid * rpw
        pltpu.sync_copy(idx_hbm.at[pl.ds(base, rpw)], idx_v)
        bufs = (rows_a, rows_b)
        g_descs = [None] * nchunks
        o_descs = [None] * nchunks
        g_descs[0] = pltpu.async_copy(
            cb_hbm.at[idx_v.at[pl.ds(0, chunk)]], bufs[0], gsem)
        for c in range(nchunks):
            if c + 1 < nchunks:
                if c >= 1:
                    o_descs[c - 1].wait()
                g_descs[c + 1] = pltpu.async_copy(
                    cb_hbm.at[idx_v.at[pl.ds((c + 1) * chunk, chunk)]],
                    bufs[(c + 1) % 2], gsem)
            g_descs[c].wait()
            o_descs[c] = pltpu.async_copy(
                bufs[c % 2], out_hbm.at[pl.ds(base + c * chunk, chunk)], osem)
        if nchunks >= 2:
            o_descs[nchunks - 2].wait()
        o_descs[nchunks - 1].wait()

    return gather_kernel(cb, idx)


def kernel(l0, l1, l2, cb0, cb1, cb2):
    levels = ((l0, cb0, 1024, 64, False), (l1, cb1, 1024, 128, False),
              (l2, cb2, 1024, 128, True))
    idxs, qs, sums = [], [], []
    for x, cb, br, chunk, with_q in levels:
        xf = x.reshape(-1, _D)
        idx, s, q = _vq_level(xf, cb, br, with_q)
        if q is None:
            q = _sc_gather(cb, idx, xf.shape[0], chunk)
        idxs.append(idx.reshape(x.shape[:-1]))
        qs.append(q.reshape(x.shape))
        sums.append(s)
    total = (
        (1.0 + _COSTS[0]) * sums[0] / l0.size
        + (1.0 + _COSTS[1]) * sums[1] / l1.size
        + (1.0 + _COSTS[2]) * sums[2] / l2.size
    )
    return (idxs[0], idxs[1], idxs[2], total, qs[0], qs[1], qs[2])
